# Initial kernel scaffold; baseline (speedup 1.0000x reference)
import jax
import jax.numpy as jnp
from jax.experimental import pallas as pl


def kernel(entity_embeddings, char_embeddings, rel_attr_embeddings, heads, rels, char_ids):
    raise NotImplementedError("write your pallas kernel here")


# speedup vs baseline: 21.8089x; 21.8089x over previous
import dataclasses
import functools

import jax
import jax.numpy as jnp
from jax import lax
from jax.experimental import pallas as pl
from jax.experimental.pallas import tpu as pltpu
from jax.experimental.pallas import tpu_sc as plsc

GAMMA = 1.0

B = 16384
D = 64
L = 100
LPAD = 112
CHAR_BINS = 144
REL_BASE = 144
NBINS = 176

NC, NS = 2, 16
NW = NC * NS
ROWS_W = B // NW
CH = 256


def _sc_gather_hist(entity_emb, heads2d, ids_pad, rels32):
    mesh = plsc.VectorSubcoreMesh(core_axis_name="c", subcore_axis_name="s")
    cp = pltpu.CompilerParams()
    for fld, val in (("needs_layout_passes", False),
                     ("use_tc_tiling_on_sc", False)):
        if fld in pltpu.CompilerParams.__dataclass_fields__:
            cp = dataclasses.replace(cp, **{fld: val})

    @functools.partial(
        pl.kernel,
        compiler_params=cp,
        out_type=(
            jax.ShapeDtypeStruct((B, D), jnp.float32),
            jax.ShapeDtypeStruct((B, NBINS), jnp.float32),
        ),
        mesh=mesh,
        scratch_types=[
            pltpu.VMEM((128,), jnp.int32),
            pltpu.VMEM((CH, D), jnp.float32),
            pltpu.VMEM((CH, LPAD), jnp.int32),
            pltpu.VMEM((ROWS_W,), jnp.int32),
            pltpu.VMEM((CH, NBINS), jnp.float32),
            pltpu.SemaphoreType.DMA,
            pltpu.SemaphoreType.DMA,
        ],
    )
    def k(ent_hbm, heads_hbm, ids_hbm, rels_hbm, h_out, cnt_out,
          idx_v, hrow_v, ids_v, rels_v, cnt_v, gsem, isem):
        wid = lax.axis_index("s") * NC + lax.axis_index("c")
        pltpu.sync_copy(rels_hbm.at[pl.ds(wid * ROWS_W, ROWS_W)], rels_v)

        ones = jnp.ones((16,), jnp.float32)
        zeros16 = jnp.zeros((16,), jnp.float32)

        for c in range(ROWS_W // CH):
            base = wid * ROWS_W + c * CH
            icp = pltpu.async_copy(ids_hbm.at[pl.ds(base, CH)], ids_v, isem)
            gcps = []
            for j in range(CH // 128):
                pltpu.sync_copy(heads_hbm.at[(base // 128) + j], idx_v)
                gcps.append(
                    pltpu.async_copy(
                        ent_hbm.at[idx_v],
                        hrow_v.at[pl.ds(j * 128, 128)],
                        gsem,
                    )
                )
            @pl.loop(0, CH)
            def _(r):
                for kk in range(NBINS // 16):
                    cnt_v[r, pl.ds(kk * 16, 16)] = zeros16

            icp.wait()

            @pl.loop(0, CH)
            def _(r):
                rows = jnp.broadcast_to(r, (16,)).astype(jnp.int32)
                for g in range(LPAD // 16):
                    ids16 = ids_v[r, pl.ds(g * 16, 16)]
                    plsc.addupdate_scatter(cnt_v, [rows, ids16], ones)

            @pl.loop(0, CH // 16)
            def _(q):
                rows = lax.iota(jnp.int32, 16) + q * 16
                cols = rels_v[pl.ds(c * CH + q * 16, 16)] + REL_BASE
                plsc.addupdate_scatter(cnt_v, [rows, cols], ones)

            pltpu.sync_copy(cnt_v, cnt_out.at[pl.ds(base, CH)])
            for g in gcps:
                g.wait()
            pltpu.sync_copy(hrow_v, h_out.at[pl.ds(base, CH)])

    return k(entity_emb, heads2d, ids_pad, rels32)


BT = 1024


def _tc_loss_body(cnt_ref, h_ref, w_ref, out_ref):
    i = pl.program_id(0)
    s = jnp.dot(cnt_ref[...], w_ref[...], preferred_element_type=jnp.float32)
    d = jnp.sum(jnp.abs(h_ref[...] - s), axis=1)
    p = jnp.sum(jnp.maximum(d + GAMMA, 0.0))

    @pl.when(i == 0)
    def _():
        out_ref[0, 0] = p

    @pl.when(i != 0)
    def _():
        out_ref[0, 0] += p


def _tc_loss(cnt, h, w):
    return pl.pallas_call(
        _tc_loss_body,
        grid=(B // BT,),
        in_specs=[
            pl.BlockSpec((BT, NBINS), lambda i: (i, 0)),
            pl.BlockSpec((BT, D), lambda i: (i, 0)),
            pl.BlockSpec((NBINS, D), lambda i: (0, 0)),
        ],
        out_specs=pl.BlockSpec(memory_space=pltpu.SMEM),
        out_shape=jax.ShapeDtypeStruct((1, 1), jnp.float32),
    )(cnt, h, w)


def kernel(entity_embeddings, char_embeddings, rel_attr_embeddings, heads, rels, char_ids):
    heads2d = heads.astype(jnp.int32).reshape(B // 128, 128)
    ids_pad = jnp.pad(
        char_ids.astype(jnp.int32), ((0, 0), (0, LPAD - L)), constant_values=128
    )
    rels32 = rels.astype(jnp.int32)
    w = jnp.concatenate(
        [
            char_embeddings,
            jnp.zeros((REL_BASE - 128, D), jnp.float32),
            -rel_attr_embeddings,
            jnp.zeros((NBINS - REL_BASE - 22, D), jnp.float32),
        ],
        axis=0,
    )
    h, cnt = _sc_gather_hist(entity_embeddings, heads2d, ids_pad, rels32)
    out = _tc_loss(cnt, h, w)
    return out[0, 0]


# speedup vs baseline: 22.0508x; 1.0111x over previous
import dataclasses
import functools

import jax
import jax.numpy as jnp
from jax import lax
from jax.experimental import pallas as pl
from jax.experimental.pallas import tpu as pltpu
from jax.experimental.pallas import tpu_sc as plsc

GAMMA = 1.0

B = 16384
D = 64
L = 100
LPAD = 112
NBINS = 128

NC, NS = 2, 16
NW = NC * NS
ROWS_W = B // NW
CH = 128


def _sc_gather_hist(ent2, heads2d, rel2, rels2d, ids_pad):
    mesh = plsc.VectorSubcoreMesh(core_axis_name="c", subcore_axis_name="s")
    cp = pltpu.CompilerParams()
    if "needs_layout_passes" in pltpu.CompilerParams.__dataclass_fields__:
        cp = dataclasses.replace(cp, needs_layout_passes=False)

    @functools.partial(
        pl.kernel,
        compiler_params=cp,
        out_type=(
            jax.ShapeDtypeStruct((B, 128), jnp.float32),
            jax.ShapeDtypeStruct((B, NBINS), jnp.float32),
        ),
        mesh=mesh,
        scratch_types=[
            pltpu.VMEM((128,), jnp.int32),
            pltpu.VMEM((128,), jnp.int32),
            pltpu.VMEM((CH, 128), jnp.float32),
            pltpu.VMEM((CH, 128), jnp.float32),
            pltpu.VMEM((CH, LPAD), jnp.int32),
            pltpu.VMEM((CH, NBINS), jnp.float32),
            pltpu.SemaphoreType.DMA,
            pltpu.SemaphoreType.DMA,
        ],
    )
    def k(ent_hbm, heads_hbm, rel_hbm, rels_hbm, ids_hbm, hr_out, cnt_out,
          hidx_v, ridx_v, ebuf_v, rbuf_v, ids_v, cnt_v, gsem, isem):
        wid = lax.axis_index("s") * NC + lax.axis_index("c")
        ones = jnp.ones((16,), jnp.float32)
        zeros16 = jnp.zeros((16,), jnp.float32)
        lastmask = lax.iota(jnp.int32, 16) < (L - (LPAD // 16 - 1) * 16)

        for c in range(ROWS_W // CH):
            base = wid * ROWS_W + c * CH
            grow = base // 128
            icp = pltpu.async_copy(ids_hbm.at[pl.ds(base, CH)], ids_v, isem)
            pltpu.sync_copy(heads_hbm.at[grow], hidx_v)
            pltpu.sync_copy(rels_hbm.at[grow], ridx_v)
            g1 = pltpu.async_copy(ent_hbm.at[hidx_v], ebuf_v, gsem)
            g2 = pltpu.async_copy(rel_hbm.at[ridx_v], rbuf_v, gsem)

            @pl.loop(0, CH)
            def _(r):
                for kk in range(NBINS // 16):
                    cnt_v[r, pl.ds(kk * 16, 16)] = zeros16

            icp.wait()

            @pl.loop(0, CH)
            def _(r):
                rows = jnp.broadcast_to(r, (16,)).astype(jnp.int32)
                for g in range(LPAD // 16):
                    ids16 = ids_v[r, pl.ds(g * 16, 16)]
                    if g == LPAD // 16 - 1:
                        plsc.addupdate_scatter(
                            cnt_v, [rows, ids16], ones, mask=lastmask)
                    else:
                        plsc.addupdate_scatter(cnt_v, [rows, ids16], ones)

            g1.wait()
            g2.wait()

            @pl.loop(0, CH)
            def _(r):
                for q in range(128 // 16):
                    sl = pl.ds(q * 16, 16)
                    ebuf_v[r, sl] = ebuf_v[r, sl] + rbuf_v[r, sl]

            pltpu.sync_copy(ebuf_v, hr_out.at[pl.ds(base, CH)])
            pltpu.sync_copy(cnt_v, cnt_out.at[pl.ds(base, CH)])

    return k(ent2, heads2d, rel2, rels2d, ids_pad)


BT = 1024


def _tc_loss_body(cnt_ref, hr_ref, par_ref, c_ref, out_ref):
    i = pl.program_id(0)
    s = jnp.dot(cnt_ref[...], c_ref[...], preferred_element_type=jnp.float32)
    hr = jnp.where(par_ref[...] > 0.5, hr_ref[:, 64:128], hr_ref[:, 0:64])
    d = jnp.sum(jnp.abs(hr - s), axis=1)
    p = jnp.sum(jnp.maximum(d + GAMMA, 0.0))

    @pl.when(i == 0)
    def _():
        out_ref[0, 0] = p

    @pl.when(i != 0)
    def _():
        out_ref[0, 0] += p


def _tc_loss(cnt, hr2, parity, cemb):
    return pl.pallas_call(
        _tc_loss_body,
        grid=(B // BT,),
        in_specs=[
            pl.BlockSpec((BT, NBINS), lambda i: (i, 0)),
            pl.BlockSpec((BT, 128), lambda i: (i, 0)),
            pl.BlockSpec((BT, 1), lambda i: (i, 0)),
            pl.BlockSpec((128, D), lambda i: (0, 0)),
        ],
        out_specs=pl.BlockSpec(memory_space=pltpu.SMEM),
        out_shape=jax.ShapeDtypeStruct((1, 1), jnp.float32),
    )(cnt, hr2, parity, cemb)


def kernel(entity_embeddings, char_embeddings, rel_attr_embeddings, heads, rels, char_ids):
    heads32 = heads.astype(jnp.int32)
    ent2 = entity_embeddings.reshape(entity_embeddings.shape[0] // 2, 128)
    heads2d = (heads32 // 2).reshape(B // 128, 128)
    parity = (heads32 % 2).astype(jnp.float32).reshape(B, 1)
    rel2 = jnp.concatenate([rel_attr_embeddings, rel_attr_embeddings], axis=1)
    rels2d = rels.astype(jnp.int32).reshape(B // 128, 128)
    ids_pad = jnp.pad(char_ids.astype(jnp.int32), ((0, 0), (0, LPAD - L)))
    hr2, cnt = _sc_gather_hist(ent2, heads2d, rel2, rels2d, ids_pad)
    out = _tc_loss(cnt, hr2, parity, char_embeddings)
    return out[0, 0]
